# baseline (speedup 1.0000x reference)
import functools

import jax
import jax.numpy as jnp
from jax import lax
from jax.experimental import pallas as pl
from jax.experimental.pallas import tpu as pltpu
from jax.experimental.pallas import tpu_sc as plsc

RB = 8
NBUF = 4


def _dyn_gather(x, idx):
    dnums = lax.GatherDimensionNumbers(
        offset_dims=(), collapsed_slice_dims=(0,), start_index_map=(0,))
    return lax.gather(x, idx[:, None], dnums, (1,),
                      mode=lax.GatherScatterMode.PROMISE_IN_BOUNDS)


def _rsqrt(x):
    i = plsc.bitcast(x, jnp.int32)
    i = jnp.int32(0x5F3759DF) - (i >> 1)
    y = plsc.bitcast(i, jnp.float32)
    for _ in range(3):
        y = y * (1.5 - 0.5 * x * y * y)
    return y


def _make_kernel(B, H, D, mesh):
    nc = mesh.num_cores
    nw = nc * mesh.num_subcores
    rows_w = B // nw
    nch = rows_w // RB
    rows_ch = RB * H
    ngrp = rows_ch // 16
    words_ch = rows_ch * D

    @functools.partial(
        pl.kernel,
        out_type=jax.ShapeDtypeStruct((B, H, D), jnp.float32),
        mesh=mesh,
        compiler_params=pltpu.CompilerParams(
            needs_layout_passes=False, use_tc_tiling_on_sc=False,
            disable_bounds_checks=True, disable_semaphore_checks=True),
        scratch_types=[
            pltpu.VMEM((NBUF, RB, H), jnp.int32),
            pltpu.VMEM((NBUF, RB, H, D), jnp.float32),
            pltpu.VMEM((D,), jnp.float32),
            pltpu.VMEM((D,), jnp.float32),
            [pltpu.SemaphoreType.DMA] * NBUF,
            [pltpu.SemaphoreType.DMA] * NBUF,
        ],
    )
    def k(ids_hbm, table_hbm, gamma_hbm, beta_hbm, out_hbm,
          idx_v, rows_v, gamma_v, beta_v, sem_g, sem_w):
        wid = lax.axis_index("s") * nc + lax.axis_index("c")
        brow0 = wid * rows_w

        pltpu.sync_copy(gamma_hbm, gamma_v)
        pltpu.sync_copy(beta_hbm, beta_v)

        iota = lax.iota(jnp.int32, 16)
        zero16 = jnp.zeros((16,), jnp.int32)
        one16 = jnp.ones((16,), jnp.int32)
        sixteen16 = jnp.full((16,), 16, jnp.int32)
        gb = [(gamma_v[pl.ds(kk * 16, 16)], beta_v[pl.ds(kk * 16, 16)])
              for kk in range(D // 16)]

        def g_copies(s, c):
            br = brow0 + c * RB
            return ([pltpu.make_async_copy(
                table_hbm.at[idx_v.at[s, r]], rows_v.at[s, r], sem_g[s])
                for r in range(RB)], br)

        def issue(s, c):
            br = brow0 + c * RB
            pltpu.sync_copy(ids_hbm.at[pl.ds(br, RB)], idx_v.at[s])
            cps, _ = g_copies(s, c)
            for cp in cps:
                cp.start()

        def wait_g(s, c):
            cps, _ = g_copies(s, c)
            for cp in cps:
                cp.wait()

        def w_copy(s, c):
            br = brow0 + c * RB
            return pltpu.make_async_copy(
                rows_v.at[s], out_hbm.at[pl.ds(br, RB)], sem_w[s])

        def compute(s):
            rows3 = rows_v.at[s]

            def group(g, _):
                gbase = g * (16 * D)

                sm = jnp.zeros((16,), jnp.float32)
                s2 = jnp.zeros((16,), jnp.float32)
                flat = iota * D + gbase
                for j in range(D):
                    v = plsc.load_gather(rows3, [zero16, zero16, flat])
                    sm = sm + v
                    s2 = s2 + v * v
                    if j + 1 < D:
                        flat = flat + one16
                mean = sm * (1.0 / D)
                var = s2 * (1.0 / D) - mean * mean
                inv = _rsqrt(var + 1e-5)
                minv = mean * inv

                idx = iota + gbase
                sel = zero16
                for r in range(16):
                    inv_r = _dyn_gather(inv, sel)
                    minv_r = _dyn_gather(minv, sel)
                    if r + 1 < 16:
                        sel = sel + one16
                    for kk in range(D // 16):
                        gk, bk = gb[kk]
                        v = plsc.load_gather(rows3, [zero16, zero16, idx])
                        y = (v * inv_r - minv_r) * gk + bk
                        plsc.store_scatter(rows3, [zero16, zero16, idx], y)
                        if r + 1 < 16 or kk + 1 < D // 16:
                            idx = idx + sixteen16
                return 0

            lax.fori_loop(0, ngrp, group, 0)

        for s in range(NBUF - 1):
            issue(s, s)

        def blk(t, _):
            for b in range(NBUF):
                c = t * NBUF + b
                wait_g(b, c)
                compute(b)
                w_copy(b, c).start()
                s = (b + NBUF - 1) % NBUF

                @pl.when(c >= 1)
                def _():
                    w_copy(s, c - 1).wait()

                @pl.when(c + NBUF - 1 < nch)
                def _():
                    issue(s, c + NBUF - 1)
            return 0

        lax.fori_loop(0, nch // NBUF, blk, 0)
        w_copy((nch - 1) % NBUF, nch - 1).wait()

    return k


def kernel(input_ids, table, gamma, beta):
    B, H = input_ids.shape
    V, D = table.shape
    if input_ids.dtype != jnp.int32:
        input_ids = input_ids.astype(jnp.int32)

    mesh = plsc.VectorSubcoreMesh(core_axis_name="c", subcore_axis_name="s")
    k = _make_kernel(B, H, D, mesh)
    return k(input_ids, table, gamma, beta)


# speedup vs baseline: 1.5993x; 1.5993x over previous
import functools

import jax
import jax.numpy as jnp
from jax import lax
from jax.experimental import pallas as pl
from jax.experimental.pallas import tpu as pltpu
from jax.experimental.pallas import tpu_sc as plsc

RB = 8
NBUF = 4
CH = 128


def _make_gather_kernel(B, H, D, mesh):
    nc = mesh.num_cores
    nw = nc * mesh.num_subcores
    rows_w = B // nw
    nch = rows_w // RB

    @functools.partial(
        pl.kernel,
        out_type=jax.ShapeDtypeStruct((B, H, D), jnp.float32),
        mesh=mesh,
        compiler_params=pltpu.CompilerParams(
            needs_layout_passes=False, use_tc_tiling_on_sc=False,
            disable_bounds_checks=True, disable_semaphore_checks=True),
        scratch_types=[
            pltpu.VMEM((NBUF, RB, H), jnp.int32),
            pltpu.VMEM((NBUF, RB, H, D), jnp.float32),
            [pltpu.SemaphoreType.DMA] * NBUF,
            [pltpu.SemaphoreType.DMA] * NBUF,
        ],
    )
    def k(ids_hbm, table_hbm, out_hbm, idx_v, rows_v, sem_g, sem_w):
        wid = lax.axis_index("s") * nc + lax.axis_index("c")
        brow0 = wid * rows_w

        def g_copies(s):
            return [pltpu.make_async_copy(
                table_hbm.at[idx_v.at[s, r]], rows_v.at[s, r], sem_g[s])
                for r in range(RB)]

        def issue(s, c):
            br = brow0 + c * RB
            pltpu.sync_copy(ids_hbm.at[pl.ds(br, RB)], idx_v.at[s])
            for cp in g_copies(s):
                cp.start()

        def wait_g(s):
            for cp in g_copies(s):
                cp.wait()

        def w_copy(s, c):
            br = brow0 + c * RB
            return pltpu.make_async_copy(
                rows_v.at[s], out_hbm.at[pl.ds(br, RB)], sem_w[s])

        for s in range(NBUF - 1):
            issue(s, s)

        def blk(t, _):
            for b in range(NBUF):
                c = t * NBUF + b
                wait_g(b)
                w_copy(b, c).start()
                s = (b + NBUF - 1) % NBUF

                @pl.when(c >= 1)
                def _():
                    w_copy(s, c - 1).wait()

                @pl.when(c + NBUF - 1 < nch)
                def _():
                    issue(s, c + NBUF - 1)
            return 0

        lax.fori_loop(0, nch // NBUF, blk, 0)
        w_copy((nch - 1) % NBUF, nch - 1).wait()

    return k


def _ln_kernel(x_ref, g_ref, b_ref, o_ref):
    x = x_ref[...]
    mean = jnp.mean(x, axis=-1, keepdims=True)
    var = jnp.mean(jnp.square(x), axis=-1, keepdims=True) - jnp.square(mean)
    inv = lax.rsqrt(var + 1e-5)
    g = g_ref[...].reshape(1, 1, -1)
    b = b_ref[...].reshape(1, 1, -1)
    o_ref[...] = (x - mean) * inv * g + b


def kernel(input_ids, table, gamma, beta):
    B, H = input_ids.shape
    V, D = table.shape
    if input_ids.dtype != jnp.int32:
        input_ids = input_ids.astype(jnp.int32)

    mesh = plsc.VectorSubcoreMesh(core_axis_name="c", subcore_axis_name="s")
    gathered = _make_gather_kernel(B, H, D, mesh)(input_ids, table)

    grid = (B // CH,)
    out = pl.pallas_call(
        _ln_kernel,
        grid=grid,
        in_specs=[
            pl.BlockSpec((CH, H, D), lambda i: (i, 0, 0)),
            pl.BlockSpec((1, D), lambda i: (0, 0)),
            pl.BlockSpec((1, D), lambda i: (0, 0)),
        ],
        out_specs=pl.BlockSpec((CH, H, D), lambda i: (i, 0, 0)),
        out_shape=jax.ShapeDtypeStruct((B, H, D), jnp.float32),
        compiler_params=pltpu.CompilerParams(
            dimension_semantics=("arbitrary",)),
    )(gathered, gamma.reshape(1, D), beta.reshape(1, D))
    return out
